# baseline (speedup 1.0000x reference)
import functools

import jax
import jax.numpy as jnp
from jax import lax
from jax.experimental import pallas as pl
from jax.experimental.pallas import tpu as pltpu

_K = 8
_EPS = 1e-10
_LANES = 128


def _block_kernel(scores_ref, g_ref, out_ref, w_ref, kh_ref, *, n_cols):
    rows = scores_ref.shape[0]
    n_chunks = n_cols // _LANES

    def ch(k):
        return pl.ds(pl.multiple_of(k * _LANES, _LANES), _LANES)

    def p1(k, macc):
        x = scores_ref[:, ch(k)] + g_ref[:, ch(k)]
        w_ref[:, ch(k)] = x
        return jnp.maximum(macc, x)

    macc = lax.fori_loop(0, n_chunks, p1,
                         jnp.full((rows, _LANES), -jnp.inf, jnp.float32),
                         unroll=8)
    c = jnp.max(macc, axis=1, keepdims=True)

    def p2(k, dacc):
        x = jnp.exp(w_ref[:, ch(k)] - c)
        w_ref[:, ch(k)] = x
        return dacc + x

    dacc = lax.fori_loop(0, n_chunks, p2,
                         jnp.zeros((rows, _LANES), jnp.float32), unroll=8)

    for t in range(_K - 1):
        rinv = 1.0 / jnp.sum(dacc, axis=1, keepdims=True)

        def pit(k, dnext, t=t, rinv=rinv):
            wv = w_ref[:, ch(k)]
            oh = wv * rinv
            khv = oh if t == 0 else kh_ref[:, ch(k)] + oh
            kh_ref[:, ch(k)] = khv
            wn = wv * jnp.maximum(1.0 - oh, _EPS)
            w_ref[:, ch(k)] = wn
            return dnext + wn

        dacc = lax.fori_loop(0, n_chunks, pit,
                             jnp.zeros((rows, _LANES), jnp.float32), unroll=8)

    rinv = 1.0 / jnp.sum(dacc, axis=1, keepdims=True)
    neg = jnp.full((rows, _LANES), -jnp.inf, jnp.float32)

    def p8(k, carry):
        accs = list(carry)
        khv = kh_ref[:, ch(k)] + w_ref[:, ch(k)] * rinv
        kh_ref[:, ch(k)] = khv
        x = khv
        for j in range(_K):
            hi = jnp.maximum(accs[j], x)
            x = jnp.minimum(accs[j], x)
            accs[j] = hi
        return tuple(accs)

    accs = lax.fori_loop(0, n_chunks, p8, (neg,) * _K, unroll=4)

    kcum = jnp.zeros((rows, 1), jnp.float32)
    tval = jnp.full((rows, 1), -jnp.inf, jnp.float32)
    work = list(accs)
    for t in range(_K):
        m = work[0]
        for j in range(1, _K):
            m = jnp.maximum(m, work[j])
        v = jnp.max(m, axis=1, keepdims=True)
        cnt = jnp.zeros((rows, 1), jnp.float32)
        for j in range(_K):
            cnt = cnt + jnp.sum((work[j] == v).astype(jnp.float32),
                                axis=1, keepdims=True)
        tval = jnp.where(kcum < 8.0, v, tval)
        kcum = kcum + cnt
        if t + 1 < _K:
            work = [jnp.where(wj == v, -jnp.inf, wj) for wj in work]

    def pout(k, ngeacc):
        ge = (kh_ref[:, ch(k)] >= tval).astype(jnp.float32)
        out_ref[:, ch(k)] = ge
        return ngeacc + ge

    ngeacc = lax.fori_loop(0, n_chunks, pout,
                           jnp.zeros((rows, _LANES), jnp.float32), unroll=8)
    n_ge = jnp.sum(ngeacc, axis=1, keepdims=True)

    @pl.when(jnp.logical_not(jnp.all(n_ge == 8.0)))
    def _():
        kh = kh_ref[...]
        col = lax.broadcasted_iota(jnp.int32, kh.shape, 1)
        gt = kh > tval
        need = 8.0 - jnp.sum(gt.astype(jnp.float32), axis=1, keepdims=True)
        base = gt
        last = jnp.full((rows, 1), -1, jnp.int32)
        for t in range(_K):
            cand = jnp.where((kh == tval) & (col > last), col, n_cols)
            j = jnp.min(cand, axis=1, keepdims=True)
            take = (float(t) < need) & (j < n_cols)
            base = base | (take & (col == j))
            last = jnp.where(take, j, last)
        out_ref[...] = base.astype(jnp.float32)


def kernel(scores, g):
    b, n = scores.shape
    rows = 8
    grid = (b // rows,)
    spec = pl.BlockSpec((rows, n), lambda i: (i, 0))
    return pl.pallas_call(
        functools.partial(_block_kernel, n_cols=n),
        grid=grid,
        in_specs=[spec, spec],
        out_specs=spec,
        out_shape=jax.ShapeDtypeStruct((b, n), jnp.float32),
        scratch_shapes=[
            pltpu.VMEM((rows, n), jnp.float32),
            pltpu.VMEM((rows, n), jnp.float32),
        ],
    )(scores, g)


# speedup vs baseline: 2.3745x; 2.3745x over previous
import functools

import jax
import jax.numpy as jnp
from jax import lax
from jax.experimental import pallas as pl
from jax.experimental.pallas import tpu as pltpu

_K = 8
_EPS = 1e-10
_LANES = 128


def _block_kernel(scores_ref, g_ref, out_ref, w_ref, kh_ref, *, n_cols):
    rows = scores_ref.shape[0]
    n_chunks = n_cols // _LANES

    macc = jnp.full((rows, _LANES), -jnp.inf, jnp.float32)
    for k in range(n_chunks):
        x = scores_ref[:, k * _LANES:(k + 1) * _LANES] + \
            g_ref[:, k * _LANES:(k + 1) * _LANES]
        w_ref[k] = x
        macc = jnp.maximum(macc, x)
    c = jnp.max(macc, axis=1, keepdims=True)

    def p2(k, dacc):
        x = jnp.exp(w_ref[k] - c)
        w_ref[k] = x
        return dacc + x

    dacc = lax.fori_loop(0, n_chunks, p2,
                         jnp.zeros((rows, _LANES), jnp.float32), unroll=8)

    for t in range(_K - 1):
        rinv = 1.0 / jnp.sum(dacc, axis=1, keepdims=True)

        def pit(k, dnext, t=t, rinv=rinv):
            wv = w_ref[k]
            oh = wv * rinv
            khv = oh if t == 0 else kh_ref[k] + oh
            kh_ref[k] = khv
            wn = wv * jnp.maximum(1.0 - oh, _EPS)
            w_ref[k] = wn
            return dnext + wn

        dacc = lax.fori_loop(0, n_chunks, pit,
                             jnp.zeros((rows, _LANES), jnp.float32), unroll=8)

    rinv = 1.0 / jnp.sum(dacc, axis=1, keepdims=True)
    neg = jnp.full((rows, _LANES), -jnp.inf, jnp.float32)

    def p8(k, carry):
        accs = list(carry)
        khv = kh_ref[k] + w_ref[k] * rinv
        kh_ref[k] = khv
        x = khv
        for j in range(_K):
            hi = jnp.maximum(accs[j], x)
            x = jnp.minimum(accs[j], x)
            accs[j] = hi
        return tuple(accs)

    accs = lax.fori_loop(0, n_chunks, p8, (neg,) * _K, unroll=4)

    kcum = jnp.zeros((rows, 1), jnp.float32)
    tval = jnp.full((rows, 1), -jnp.inf, jnp.float32)
    work = list(accs)
    for t in range(_K):
        m = work[0]
        for j in range(1, _K):
            m = jnp.maximum(m, work[j])
        v = jnp.max(m, axis=1, keepdims=True)
        cnt = jnp.zeros((rows, 1), jnp.float32)
        for j in range(_K):
            cnt = cnt + jnp.sum((work[j] == v).astype(jnp.float32),
                                axis=1, keepdims=True)
        tval = jnp.where(kcum < 8.0, v, tval)
        kcum = kcum + cnt
        if t + 1 < _K:
            work = [jnp.where(wj == v, -jnp.inf, wj) for wj in work]

    ngeacc = jnp.zeros((rows, _LANES), jnp.float32)
    for k in range(n_chunks):
        ge = (kh_ref[k] >= tval).astype(jnp.float32)
        out_ref[:, k * _LANES:(k + 1) * _LANES] = ge
        ngeacc = ngeacc + ge
    n_ge = jnp.sum(ngeacc, axis=1, keepdims=True)

    @pl.when(jnp.logical_not(jnp.all(n_ge == 8.0)))
    def _():
        kh = kh_ref[...]
        col = (lax.broadcasted_iota(jnp.int32, kh.shape, 0) * _LANES +
               lax.broadcasted_iota(jnp.int32, kh.shape, 2))
        tv3 = tval[jnp.newaxis]

        def redmin(x):
            return jnp.min(jnp.min(x, axis=2), axis=0)[jnp.newaxis, :,
                                                       jnp.newaxis]

        def redsum(x):
            return jnp.sum(jnp.sum(x, axis=2), axis=0)[jnp.newaxis, :,
                                                       jnp.newaxis]

        gt = kh > tv3
        need = 8.0 - redsum(gt.astype(jnp.float32))
        base = gt
        last = jnp.full((1, rows, 1), -1, jnp.int32)
        for t in range(_K):
            cand = jnp.where((kh == tv3) & (col > last), col, n_cols)
            j = redmin(cand)
            take = (float(t) < need) & (j < n_cols)
            base = base | (take & (col == j))
            last = jnp.where(take, j, last)
        basef = base.astype(jnp.float32)
        for k in range(n_chunks):
            out_ref[:, k * _LANES:(k + 1) * _LANES] = basef[k]


def kernel(scores, g):
    b, n = scores.shape
    rows = 8
    grid = (b // rows,)
    spec = pl.BlockSpec((rows, n), lambda i: (i, 0))
    return pl.pallas_call(
        functools.partial(_block_kernel, n_cols=n),
        grid=grid,
        in_specs=[spec, spec],
        out_specs=spec,
        out_shape=jax.ShapeDtypeStruct((b, n), jnp.float32),
        scratch_shapes=[
            pltpu.VMEM((n // _LANES, rows, _LANES), jnp.float32),
            pltpu.VMEM((n // _LANES, rows, _LANES), jnp.float32),
        ],
    )(scores, g)


# speedup vs baseline: 2.6341x; 1.1093x over previous
import functools

import jax
import jax.numpy as jnp
from jax import lax
from jax.experimental import pallas as pl
from jax.experimental.pallas import tpu as pltpu

_K = 8
_EPS = 1e-10
_LANES = 128
_G = 8
_NETS = 4


def _block_kernel(scores_ref, g_ref, out_ref, w_ref, kh_ref, *, n_cols):
    rows = scores_ref.shape[0]
    n_chunks = n_cols // _LANES
    n_groups = n_chunks // _G

    neg = jnp.full((rows, _LANES), -jnp.inf, jnp.float32)
    maccs = [neg] * _G
    for k in range(n_chunks):
        x = scores_ref[:, k * _LANES:(k + 1) * _LANES] + \
            g_ref[:, k * _LANES:(k + 1) * _LANES]
        w_ref[k] = x
        maccs[k % _G] = jnp.maximum(maccs[k % _G], x)
    m = maccs[0]
    for j in range(1, _G):
        m = jnp.maximum(m, maccs[j])
    c = jnp.max(m, axis=1, keepdims=True)

    zero = jnp.zeros((rows, _LANES), jnp.float32)

    def p2(i, daccs):
        daccs = list(daccs)
        for j in range(_G):
            x = jnp.exp(w_ref[i * _G + j] - c)
            w_ref[i * _G + j] = x
            daccs[j] = daccs[j] + x
        return tuple(daccs)

    daccs = lax.fori_loop(0, n_groups, p2, (zero,) * _G)

    def dsum(daccs):
        d = daccs[0]
        for j in range(1, _G):
            d = d + daccs[j]
        return 1.0 / jnp.sum(d, axis=1, keepdims=True)

    for t in range(_K - 1):
        rinv = dsum(daccs)

        def pit(i, daccs, t=t, rinv=rinv):
            daccs = list(daccs)
            for j in range(_G):
                k = i * _G + j
                wv = w_ref[k]
                oh = wv * rinv
                khv = oh if t == 0 else kh_ref[k] + oh
                kh_ref[k] = khv
                wn = wv * jnp.maximum(1.0 - oh, _EPS)
                w_ref[k] = wn
                daccs[j] = daccs[j] + wn
            return tuple(daccs)

        daccs = lax.fori_loop(0, n_groups, pit, (zero,) * _G)

    rinv = dsum(daccs)

    def p8(i, carry):
        nets = [list(carry[a * _K:(a + 1) * _K]) for a in range(_NETS)]
        for j in range(_G):
            k = i * _G + j
            khv = kh_ref[k] + w_ref[k] * rinv
            kh_ref[k] = khv
            net = nets[j % _NETS]
            x = khv
            for lvl in range(_K):
                hi = jnp.maximum(net[lvl], x)
                x = jnp.minimum(net[lvl], x)
                net[lvl] = hi
        return tuple(v for net in nets for v in net)

    flat = lax.fori_loop(0, n_groups, p8, (neg,) * (_K * _NETS))
    cands = list(flat)

    kcum = jnp.zeros((rows, 1), jnp.float32)
    tval = jnp.full((rows, 1), -jnp.inf, jnp.float32)
    for t in range(_K):
        m = cands[0]
        for wj in cands[1:]:
            m = jnp.maximum(m, wj)
        v = jnp.max(m, axis=1, keepdims=True)
        cnt = jnp.zeros((rows, 1), jnp.float32)
        for wj in cands:
            cnt = cnt + jnp.sum((wj == v).astype(jnp.float32),
                                axis=1, keepdims=True)
        tval = jnp.where(kcum < 8.0, v, tval)
        kcum = kcum + cnt
        if t + 1 < _K:
            cands = [jnp.where(wj == v, -jnp.inf, wj) for wj in cands]

    ngeaccs = [zero] * _G
    for k in range(n_chunks):
        ge = (kh_ref[k] >= tval).astype(jnp.float32)
        out_ref[:, k * _LANES:(k + 1) * _LANES] = ge
        ngeaccs[k % _G] = ngeaccs[k % _G] + ge
    ng = ngeaccs[0]
    for j in range(1, _G):
        ng = ng + ngeaccs[j]
    n_ge = jnp.sum(ng, axis=1, keepdims=True)

    @pl.when(jnp.logical_not(jnp.all(n_ge == 8.0)))
    def _():
        kh = kh_ref[...]
        col = (lax.broadcasted_iota(jnp.int32, kh.shape, 0) * _LANES +
               lax.broadcasted_iota(jnp.int32, kh.shape, 2))
        tv3 = tval[jnp.newaxis]

        def redmin(x):
            return jnp.min(jnp.min(x, axis=2), axis=0)[jnp.newaxis, :,
                                                       jnp.newaxis]

        def redsum(x):
            return jnp.sum(jnp.sum(x, axis=2), axis=0)[jnp.newaxis, :,
                                                       jnp.newaxis]

        gt = kh > tv3
        need = 8.0 - redsum(gt.astype(jnp.float32))
        base = gt
        last = jnp.full((1, rows, 1), -1, jnp.int32)
        for t in range(_K):
            cand = jnp.where((kh == tv3) & (col > last), col, n_cols)
            j = redmin(cand)
            take = (float(t) < need) & (j < n_cols)
            base = base | (take & (col == j))
            last = jnp.where(take, j, last)
        basef = base.astype(jnp.float32)
        for k in range(n_chunks):
            out_ref[:, k * _LANES:(k + 1) * _LANES] = basef[k]


def kernel(scores, g):
    b, n = scores.shape
    rows = 8
    grid = (b // rows,)
    spec = pl.BlockSpec((rows, n), lambda i: (i, 0))
    return pl.pallas_call(
        functools.partial(_block_kernel, n_cols=n),
        grid=grid,
        in_specs=[spec, spec],
        out_specs=spec,
        out_shape=jax.ShapeDtypeStruct((b, n), jnp.float32),
        scratch_shapes=[
            pltpu.VMEM((n // _LANES, rows, _LANES), jnp.float32),
            pltpu.VMEM((n // _LANES, rows, _LANES), jnp.float32),
        ],
    )(scores, g)


# speedup vs baseline: 4.2283x; 1.6052x over previous
import functools

import jax
import jax.numpy as jnp
from jax import lax
from jax.experimental import pallas as pl

_K = 8
_EPS = 1e-10
_LANES = 128
_ROWS = 16


def _block_kernel(scores_ref, g_ref, out_ref, *, n_cols):
    s = scores_ref[...] + g_ref[...]
    c = jnp.max(s, axis=1, keepdims=True)
    w = jnp.exp(s - c)
    kh = jnp.zeros_like(w)
    for t in range(_K):
        d = jnp.sum(w, axis=1, keepdims=True)
        oh = w * (1.0 / d)
        kh = kh + oh
        if t + 1 < _K:
            w = w * jnp.maximum(1.0 - oh, _EPS)

    rows = kh.shape[0]
    n_chunks = n_cols // _LANES

    neg = jnp.full((rows, _LANES), -jnp.inf, jnp.float32)
    accs = [neg] * _K
    for k in range(n_chunks):
        x = kh[:, k * _LANES:(k + 1) * _LANES]
        for j in range(_K):
            hi = jnp.maximum(accs[j], x)
            x = jnp.minimum(accs[j], x)
            accs[j] = hi

    kcum = jnp.zeros((rows, 1), jnp.float32)
    tval = jnp.full((rows, 1), -jnp.inf, jnp.float32)
    work = list(accs)
    for t in range(_K):
        m = work[0]
        for j in range(1, _K):
            m = jnp.maximum(m, work[j])
        v = jnp.max(m, axis=1, keepdims=True)
        eqs = (work[0] == v).astype(jnp.float32)
        for j in range(1, _K):
            eqs = eqs + (work[j] == v).astype(jnp.float32)
        cnt = jnp.sum(eqs, axis=1, keepdims=True)
        tval = jnp.where(kcum < 8.0, v, tval)
        kcum = kcum + cnt
        if t + 1 < _K:
            work = [jnp.where(wj == v, -jnp.inf, wj) for wj in work]

    ge = kh >= tval
    n_ge = jnp.sum(ge.astype(jnp.float32), axis=1, keepdims=True)
    exact = jnp.all(n_ge == 8.0)

    @pl.when(exact)
    def _():
        out_ref[...] = ge.astype(jnp.float32)

    @pl.when(jnp.logical_not(exact))
    def _():
        col = lax.broadcasted_iota(jnp.int32, kh.shape, 1)
        gt = kh > tval
        need = 8.0 - jnp.sum(gt.astype(jnp.float32), axis=1, keepdims=True)
        base = gt
        last = jnp.full((rows, 1), -1, jnp.int32)
        for t in range(_K):
            cand = jnp.where((kh == tval) & (col > last), col, n_cols)
            j = jnp.min(cand, axis=1, keepdims=True)
            take = (float(t) < need) & (j < n_cols)
            base = base | (take & (col == j))
            last = jnp.where(take, j, last)
        out_ref[...] = base.astype(jnp.float32)


def kernel(scores, g):
    b, n = scores.shape
    rows = _ROWS
    grid = (b // rows,)
    spec = pl.BlockSpec((rows, n), lambda i: (i, 0))
    return pl.pallas_call(
        functools.partial(_block_kernel, n_cols=n),
        grid=grid,
        in_specs=[spec, spec],
        out_specs=spec,
        out_shape=jax.ShapeDtypeStruct((b, n), jnp.float32),
    )(scores, g)


# speedup vs baseline: 4.3871x; 1.0376x over previous
import functools

import jax
import jax.numpy as jnp
from jax import lax
from jax.experimental import pallas as pl

_K = 8
_EPS = 1e-10
_LANES = 128
_ROWS = 32


def _block_kernel(scores_ref, g_ref, out_ref, *, n_cols):
    s = scores_ref[...] + g_ref[...]
    c = jnp.max(s, axis=1, keepdims=True)
    w = jnp.exp(s - c)
    kh = jnp.zeros_like(w)
    for t in range(_K):
        d = jnp.sum(w, axis=1, keepdims=True)
        oh = w * (1.0 / d)
        kh = kh + oh
        if t + 1 < _K:
            w = w * jnp.maximum(1.0 - oh, _EPS)

    rows = kh.shape[0]
    n_chunks = n_cols // _LANES

    neg = jnp.full((rows, _LANES), -jnp.inf, jnp.float32)
    accs = [neg] * _K
    for k in range(n_chunks):
        x = kh[:, k * _LANES:(k + 1) * _LANES]
        for j in range(_K):
            hi = jnp.maximum(accs[j], x)
            x = jnp.minimum(accs[j], x)
            accs[j] = hi

    kcum = jnp.zeros((rows, 1), jnp.float32)
    tval = jnp.full((rows, 1), -jnp.inf, jnp.float32)
    work = list(accs)
    for t in range(_K):
        m = work[0]
        for j in range(1, _K):
            m = jnp.maximum(m, work[j])
        v = jnp.max(m, axis=1, keepdims=True)
        eqs = (work[0] == v).astype(jnp.float32)
        for j in range(1, _K):
            eqs = eqs + (work[j] == v).astype(jnp.float32)
        cnt = jnp.sum(eqs, axis=1, keepdims=True)
        tval = jnp.where(kcum < 8.0, v, tval)
        kcum = kcum + cnt
        if t + 1 < _K:
            work = [jnp.where(wj == v, -jnp.inf, wj) for wj in work]

    ge = kh >= tval
    n_ge = jnp.sum(ge.astype(jnp.float32), axis=1, keepdims=True)
    exact = jnp.all(n_ge == 8.0)

    @pl.when(exact)
    def _():
        out_ref[...] = ge.astype(jnp.float32)

    @pl.when(jnp.logical_not(exact))
    def _():
        col = lax.broadcasted_iota(jnp.int32, kh.shape, 1)
        gt = kh > tval
        need = 8.0 - jnp.sum(gt.astype(jnp.float32), axis=1, keepdims=True)
        base = gt
        last = jnp.full((rows, 1), -1, jnp.int32)
        for t in range(_K):
            cand = jnp.where((kh == tval) & (col > last), col, n_cols)
            j = jnp.min(cand, axis=1, keepdims=True)
            take = (float(t) < need) & (j < n_cols)
            base = base | (take & (col == j))
            last = jnp.where(take, j, last)
        out_ref[...] = base.astype(jnp.float32)


def kernel(scores, g):
    b, n = scores.shape
    rows = _ROWS
    grid = (b // rows,)
    spec = pl.BlockSpec((rows, n), lambda i: (i, 0))
    return pl.pallas_call(
        functools.partial(_block_kernel, n_cols=n),
        grid=grid,
        in_specs=[spec, spec],
        out_specs=spec,
        out_shape=jax.ShapeDtypeStruct((b, n), jnp.float32),
    )(scores, g)
